# baseline (speedup 1.0000x reference)
import functools

import jax
import jax.numpy as jnp
from jax import lax
from jax.experimental import pallas as pl
from jax.experimental.pallas import tpu as pltpu
from jax.experimental.pallas import tpu_sc as plsc

T = 2048
D = 1024
R = 16
BITS = 4
MEM = 128
C = R * BITS
ALPHA = 1 << BITS
NO = 2

NC = 2
NS = 16
NW = NC * NS
TPW = T // NW
SUB = 16
NSUB = TPW // SUB
ROWS = SUB * R

KB = 512
NKB = (R * MEM) // KB


TB = 1024
NTB = T // TB


def _index_body(x_ref, wq_ref, gidx_ref, codes_ref, carry_ref):
    tb = pl.program_id(0)
    q = lax.dot_general(x_ref[0], wq_ref[...],
                        (((1,), (1,)), ((), ())))
    bits = (q > 0).astype(jnp.float32)
    c_i = lax.broadcasted_iota(jnp.int32, (C, R), 0)
    r_i = lax.broadcasted_iota(jnp.int32, (C, R), 1)
    sel = jnp.where(c_i // BITS == r_i, 1 << (c_i % BITS), 0).astype(jnp.float32)
    codes = jnp.dot(bits, sel).astype(jnp.int32)
    t_i = lax.broadcasted_iota(jnp.int32, (TB, R), 0)
    r_t = lax.broadcasted_iota(jnp.int32, (TB, R), 1)
    prev = jnp.where(tb > 0, carry_ref[0:2], jnp.zeros((2, R), jnp.int32))
    c0 = codes
    c1 = jnp.where(t_i >= 1, pltpu.roll(codes, 1, 0), prev[1][None])
    c2 = jnp.where(t_i >= 2, pltpu.roll(codes, 2, 0),
                   jnp.where(t_i == 1, prev[1][None], prev[0][None]))
    carry_ref[0:2] = lax.slice(codes, (TB - 2, 0), (TB, R))
    codes_ref[...] = codes
    gidx_ref[0] = r_t * ALPHA**2 + c1 + ALPHA * c0
    gidx_ref[1] = r_t * ALPHA**3 + c2 + ALPHA * c1 + ALPHA**2 * c0


def _p1_body(t1_ref, wo1_ref, codes_ref, p1_ref, oh_ref):
    for r in range(R):
        p1_ref[pl.ds(r * ALPHA, ALPHA), :] = jnp.dot(
            t1_ref[pl.ds(r * ALPHA, ALPHA), :], wo1_ref[0, r],
            preferred_element_type=jnp.float32)
    g_r = lax.broadcasted_iota(jnp.int32, (R, R * ALPHA), 0)
    g_c = lax.broadcasted_iota(jnp.int32, (R, R * ALPHA), 1)
    erep = jnp.where(g_c // ALPHA == g_r, 1.0, 0.0)
    c_rep = jnp.dot(codes_ref[...].astype(jnp.float32), erep)
    a_i = lax.broadcasted_iota(jnp.int32, (T, R * ALPHA), 1) % ALPHA
    oh_ref[...] = (c_rep.astype(jnp.int32) == a_i).astype(jnp.float32)


def _sc_gather_body(t2, t3, gidx, out,
                    idx_a, idx_b, rows_a, rows_b, sem_a, sem_b):
    wid = lax.axis_index("s") * NC + lax.axis_index("c")
    tabs = (t2, t3)
    idx_v = (idx_a, idx_b)
    rows_v = (rows_a, rows_b)
    sems = (sem_a, sem_b)
    work = [(s, n) for s in range(NSUB) for n in range(NO)]

    def start(i, b):
        s, n = work[i]
        base = n * T * R + (wid * TPW + s * SUB) * R
        pltpu.sync_copy(gidx.at[pl.ds(base, ROWS)], idx_v[b])
        return pltpu.async_copy(tabs[n].at[idx_v[b]], rows_v[b], sems[b])

    pending = {0: start(0, 0)}
    for i, (s, n) in enumerate(work):
        b = i % 2
        if i + 1 < len(work):
            pending[i + 1] = start(i + 1, 1 - b)
        pending.pop(i).wait()
        t0 = wid * TPW + s * SUB
        pltpu.sync_copy(rows_v[b].reshape(SUB, R * MEM),
                        out.at[pl.ds(n * T + t0, SUB)])


def _mm_body(oh_ref, p1_ref, flat_ref, wo_ref, o_ref):
    n = pl.program_id(0)
    k = pl.program_id(1)

    @pl.when((n == 0) & (k == 0))
    def _():
        o_ref[...] = jnp.dot(oh_ref[...], p1_ref[...],
                             preferred_element_type=jnp.float32)

    a = flat_ref[0]
    t_i = lax.broadcasted_iota(jnp.int32, a.shape, 0)
    a = jnp.where(t_i >= n + 1, a, 0.0)
    o_ref[...] += jnp.dot(a, wo_ref[0], preferred_element_type=jnp.float32)


def kernel(x, Wq, table_1, table_2, table_3, Wo):
    gidx, codes = pl.pallas_call(
        _index_body,
        grid=(NTB,),
        in_specs=[
            pl.BlockSpec((1, TB, D), lambda tb: (0, tb, 0)),
            pl.BlockSpec((C, D), lambda tb: (0, 0)),
        ],
        out_specs=(pl.BlockSpec((NO, TB, R), lambda tb: (0, tb, 0)),
                   pl.BlockSpec((TB, R), lambda tb: (tb, 0))),
        out_shape=(jax.ShapeDtypeStruct((NO, T, R), jnp.int32),
                   jax.ShapeDtypeStruct((T, R), jnp.int32)),
        scratch_shapes=[pltpu.VMEM((8, R), jnp.int32)],
        compiler_params=pltpu.CompilerParams(
            dimension_semantics=("arbitrary",),
        ),
    )(x, Wq.T)
    gflat = gidx.reshape(NO * T * R)

    wo4 = Wo.reshape(3, R, MEM, D)
    p1, oh = pl.pallas_call(
        _p1_body,
        grid=(1,),
        in_specs=[
            pl.BlockSpec((R * ALPHA, MEM), lambda i: (0, 0)),
            pl.BlockSpec((1, R, MEM, D), lambda i: (0, 0, 0, 0)),
            pl.BlockSpec((T, R), lambda i: (0, 0)),
        ],
        out_specs=(pl.BlockSpec((R * ALPHA, D), lambda i: (0, 0)),
                   pl.BlockSpec((T, R * ALPHA), lambda i: (0, 0))),
        out_shape=(jax.ShapeDtypeStruct((R * ALPHA, D), jnp.float32),
                   jax.ShapeDtypeStruct((T, R * ALPHA), jnp.float32)),
    )(table_1, wo4, codes)

    mesh = plsc.VectorSubcoreMesh(core_axis_name="c", subcore_axis_name="s")
    sc_scratch = [
        pltpu.VMEM((ROWS,), jnp.int32),
        pltpu.VMEM((ROWS,), jnp.int32),
        pltpu.VMEM((ROWS, MEM), jnp.float32),
        pltpu.VMEM((ROWS, MEM), jnp.float32),
        pltpu.SemaphoreType.DMA,
        pltpu.SemaphoreType.DMA,
    ]

    sc_gather = functools.partial(
        pl.kernel,
        out_type=jax.ShapeDtypeStruct((NO * T, R * MEM), jnp.float32),
        mesh=mesh,
        scratch_types=sc_scratch,
    )(_sc_gather_body)
    rows = sc_gather(table_2, table_3, gflat)
    flat = rows.reshape(NO, T, R * MEM)

    wo3 = Wo.reshape(3, R * MEM, D)
    out = pl.pallas_call(
        _mm_body,
        grid=(NO, NKB),
        in_specs=[
            pl.BlockSpec((T, R * ALPHA), lambda n, k: (0, 0)),
            pl.BlockSpec((R * ALPHA, D), lambda n, k: (0, 0)),
            pl.BlockSpec((1, T, KB), lambda n, k: (n, 0, k)),
            pl.BlockSpec((1, KB, D), lambda n, k: (n + 1, k, 0)),
        ],
        out_specs=pl.BlockSpec((T, D), lambda n, k: (0, 0)),
        out_shape=jax.ShapeDtypeStruct((T, D), jnp.float32),
        compiler_params=pltpu.CompilerParams(
            dimension_semantics=("arbitrary", "arbitrary"),
        ),
    )(oh, p1, flat, wo3)

    return out.reshape(x.shape[0], T, D)


# speedup vs baseline: 1.0102x; 1.0102x over previous
import functools

import jax
import jax.numpy as jnp
from jax import lax
from jax.experimental import pallas as pl
from jax.experimental.pallas import tpu as pltpu
from jax.experimental.pallas import tpu_sc as plsc

T = 2048
D = 1024
R = 16
BITS = 4
MEM = 128
C = R * BITS
ALPHA = 1 << BITS
NO = 2

NC = 2
NS = 16
NW = NC * NS
TPW = T // NW
SUB = 16
NSUB = TPW // SUB
ROWS = SUB * R

KB = 512
NKB = (R * MEM) // KB


TB = 1024
NTB = T // TB


def _index_body(x_ref, wq_ref, gidx_ref, codes_ref, carry_ref):
    tb = pl.program_id(0)
    q = lax.dot_general(x_ref[0], wq_ref[...],
                        (((1,), (1,)), ((), ())))
    bits = (q > 0).astype(jnp.float32)
    c_i = lax.broadcasted_iota(jnp.int32, (C, R), 0)
    r_i = lax.broadcasted_iota(jnp.int32, (C, R), 1)
    sel = jnp.where(c_i // BITS == r_i, 1 << (c_i % BITS), 0).astype(jnp.float32)
    codes = jnp.dot(bits, sel).astype(jnp.int32)
    t_i = lax.broadcasted_iota(jnp.int32, (TB, R), 0)
    r_t = lax.broadcasted_iota(jnp.int32, (TB, R), 1)
    prev = jnp.where(tb > 0, carry_ref[0:2], jnp.zeros((2, R), jnp.int32))
    c0 = codes
    c1 = jnp.where(t_i >= 1, pltpu.roll(codes, 1, 0), prev[1][None])
    c2 = jnp.where(t_i >= 2, pltpu.roll(codes, 2, 0),
                   jnp.where(t_i == 1, prev[1][None], prev[0][None]))
    carry_ref[0:2] = lax.slice(codes, (TB - 2, 0), (TB, R))
    codes_ref[...] = codes
    gidx_ref[0] = r_t * ALPHA**2 + c1 + ALPHA * c0
    gidx_ref[1] = r_t * ALPHA**3 + c2 + ALPHA * c1 + ALPHA**2 * c0


def _p1_body(t1_ref, wo1_ref, p1_ref):
    for r in range(R):
        p1_ref[pl.ds(r * ALPHA, ALPHA), :] = jnp.dot(
            t1_ref[pl.ds(r * ALPHA, ALPHA), :], wo1_ref[0, r],
            preferred_element_type=jnp.float32)


def _sc_gather_body(t2, t3, gidx, out,
                    i0, i1, i2, i3, i4, i5, i6, i7,
                    rows_a, rows_b, sem_i, sem_a, sem_b):
    wid = lax.axis_index("s") * NC + lax.axis_index("c")
    tabs = (t2, t3)
    idxs = (i0, i1, i2, i3, i4, i5, i6, i7)
    rows_v = (rows_a, rows_b)
    sems = (sem_a, sem_b)
    work = [(s, n) for s in range(NSUB) for n in range(NO)]
    cps = []
    for i, (s, n) in enumerate(work):
        base = n * T * R + (wid * TPW + s * SUB) * R
        cps.append(pltpu.async_copy(gidx.at[pl.ds(base, ROWS)], idxs[i], sem_i))
    for c in cps:
        c.wait()

    def start(i, b):
        s, n = work[i]
        return pltpu.async_copy(tabs[n].at[idxs[i]], rows_v[b], sems[b])

    pending = {0: start(0, 0)}
    for i, (s, n) in enumerate(work):
        b = i % 2
        if i + 1 < len(work):
            pending[i + 1] = start(i + 1, 1 - b)
        pending.pop(i).wait()
        t0 = wid * TPW + s * SUB
        pltpu.sync_copy(rows_v[b].reshape(SUB, R * MEM),
                        out.at[pl.ds(n * T + t0, SUB)])


def _mm_body(codes_ref, p1_ref, flat_ref, wo_ref, o_ref):
    n = pl.program_id(0)
    k = pl.program_id(1)

    @pl.when((n == 0) & (k == 0))
    def _():
        g_r = lax.broadcasted_iota(jnp.int32, (R, R * ALPHA), 0)
        g_c = lax.broadcasted_iota(jnp.int32, (R, R * ALPHA), 1)
        erep = jnp.where(g_c // ALPHA == g_r, 1.0, 0.0)
        c_rep = jnp.dot(codes_ref[...].astype(jnp.float32), erep)
        a_i = lax.broadcasted_iota(jnp.int32, (T, R * ALPHA), 1) % ALPHA
        onehot = (c_rep.astype(jnp.int32) == a_i).astype(jnp.float32)
        o_ref[...] = jnp.dot(onehot, p1_ref[...],
                             preferred_element_type=jnp.float32)

    a = flat_ref[0]
    t_i = lax.broadcasted_iota(jnp.int32, a.shape, 0)
    a = jnp.where(t_i >= n + 1, a, 0.0)
    o_ref[...] += jnp.dot(a, wo_ref[0], preferred_element_type=jnp.float32)


def kernel(x, Wq, table_1, table_2, table_3, Wo):
    gidx, codes = pl.pallas_call(
        _index_body,
        grid=(NTB,),
        in_specs=[
            pl.BlockSpec((1, TB, D), lambda tb: (0, tb, 0)),
            pl.BlockSpec((C, D), lambda tb: (0, 0)),
        ],
        out_specs=(pl.BlockSpec((NO, TB, R), lambda tb: (0, tb, 0)),
                   pl.BlockSpec((TB, R), lambda tb: (tb, 0))),
        out_shape=(jax.ShapeDtypeStruct((NO, T, R), jnp.int32),
                   jax.ShapeDtypeStruct((T, R), jnp.int32)),
        scratch_shapes=[pltpu.VMEM((8, R), jnp.int32)],
        compiler_params=pltpu.CompilerParams(
            dimension_semantics=("arbitrary",),
        ),
    )(x, Wq.T)
    gflat = gidx.reshape(NO * T * R)

    wo4 = Wo.reshape(3, R, MEM, D)
    p1 = pl.pallas_call(
        _p1_body,
        grid=(1,),
        in_specs=[
            pl.BlockSpec((R * ALPHA, MEM), lambda i: (0, 0)),
            pl.BlockSpec((1, R, MEM, D), lambda i: (0, 0, 0, 0)),
        ],
        out_specs=pl.BlockSpec((R * ALPHA, D), lambda i: (0, 0)),
        out_shape=jax.ShapeDtypeStruct((R * ALPHA, D), jnp.float32),
    )(table_1, wo4)

    mesh = plsc.VectorSubcoreMesh(core_axis_name="c", subcore_axis_name="s")
    sc_scratch = (
        [pltpu.VMEM((ROWS,), jnp.int32) for _ in range(NO * NSUB)]
        + [
            pltpu.VMEM((ROWS, MEM), jnp.float32),
            pltpu.VMEM((ROWS, MEM), jnp.float32),
            pltpu.SemaphoreType.DMA,
            pltpu.SemaphoreType.DMA,
            pltpu.SemaphoreType.DMA,
        ]
    )

    sc_gather = functools.partial(
        pl.kernel,
        out_type=jax.ShapeDtypeStruct((NO * T, R * MEM), jnp.float32),
        mesh=mesh,
        scratch_types=sc_scratch,
    )(_sc_gather_body)
    rows = sc_gather(table_2, table_3, gflat)
    flat = rows.reshape(NO, T, R * MEM)

    wo3 = Wo.reshape(3, R * MEM, D)
    out = pl.pallas_call(
        _mm_body,
        grid=(NO, NKB),
        in_specs=[
            pl.BlockSpec((T, R), lambda n, k: (0, 0)),
            pl.BlockSpec((R * ALPHA, D), lambda n, k: (0, 0)),
            pl.BlockSpec((1, T, KB), lambda n, k: (n, 0, k)),
            pl.BlockSpec((1, KB, D), lambda n, k: (n + 1, k, 0)),
        ],
        out_specs=pl.BlockSpec((T, D), lambda n, k: (0, 0)),
        out_shape=jax.ShapeDtypeStruct((T, D), jnp.float32),
        compiler_params=pltpu.CompilerParams(
            dimension_semantics=("arbitrary", "arbitrary"),
        ),
    )(codes, p1, flat, wo3)

    return out.reshape(x.shape[0], T, D)


# speedup vs baseline: 1.0136x; 1.0034x over previous
import functools

import jax
import jax.numpy as jnp
from jax import lax
from jax.experimental import pallas as pl
from jax.experimental.pallas import tpu as pltpu
from jax.experimental.pallas import tpu_sc as plsc

T = 2048
D = 1024
R = 16
BITS = 4
MEM = 128
C = R * BITS
ALPHA = 1 << BITS
NO = 2

NC = 2
NS = 16
NW = NC * NS
TPW = T // NW
SUB = 16
NSUB = TPW // SUB
ROWS = SUB * R

KB = 512
NKB = (R * MEM) // KB


TB = 1024
NTB = T // TB


def _index_body(x_ref, wq_ref, gidx_ref, codes_ref, carry_ref):
    tb = pl.program_id(0)
    q = lax.dot_general(x_ref[0], wq_ref[...],
                        (((1,), (1,)), ((), ())))
    bits = (q > 0).astype(jnp.float32)
    c_i = lax.broadcasted_iota(jnp.int32, (C, R), 0)
    r_i = lax.broadcasted_iota(jnp.int32, (C, R), 1)
    sel = jnp.where(c_i // BITS == r_i, 1 << (c_i % BITS), 0).astype(jnp.float32)
    codes = jnp.dot(bits, sel).astype(jnp.int32)
    t_i = lax.broadcasted_iota(jnp.int32, (TB, R), 0)
    r_t = lax.broadcasted_iota(jnp.int32, (TB, R), 1)
    prev = jnp.where(tb > 0, carry_ref[0:2], jnp.zeros((2, R), jnp.int32))
    c0 = codes
    c1 = jnp.where(t_i >= 1, pltpu.roll(codes, 1, 0), prev[1][None])
    c2 = jnp.where(t_i >= 2, pltpu.roll(codes, 2, 0),
                   jnp.where(t_i == 1, prev[1][None], prev[0][None]))
    carry_ref[0:2] = lax.slice(codes, (TB - 2, 0), (TB, R))
    codes_ref[...] = codes
    gidx_ref[0] = r_t * ALPHA**2 + c1 + ALPHA * c0
    gidx_ref[1] = r_t * ALPHA**3 + c2 + ALPHA * c1 + ALPHA**2 * c0


def _p1_body(t1_ref, wo1_ref, p1_ref):
    for r in range(R):
        p1_ref[pl.ds(r * ALPHA, ALPHA), :] = jnp.dot(
            t1_ref[pl.ds(r * ALPHA, ALPHA), :], wo1_ref[0, r],
            preferred_element_type=jnp.float32)


def _sc_gather_body(t2, t3, gidx, out,
                    idx_a, idx_b, rows_a, rows_b, sem_a, sem_b):
    wid = lax.axis_index("s") * NC + lax.axis_index("c")
    tabs = (t2, t3)
    idx_v = (idx_a, idx_b)
    rows_v = (rows_a, rows_b)
    sems = (sem_a, sem_b)
    work = [(s, n) for s in range(NSUB) for n in range(NO)]

    def start(i, b):
        s, n = work[i]
        base = n * T * R + (wid * TPW + s * SUB) * R
        pltpu.sync_copy(gidx.at[pl.ds(base, ROWS)], idx_v[b])
        return pltpu.async_copy(tabs[n].at[idx_v[b]], rows_v[b], sems[b])

    pending = {0: start(0, 0)}
    for i, (s, n) in enumerate(work):
        b = i % 2
        if i + 1 < len(work):
            pending[i + 1] = start(i + 1, 1 - b)
        pending.pop(i).wait()
        t0 = wid * TPW + s * SUB
        pltpu.sync_copy(rows_v[b].reshape(SUB, R * MEM),
                        out.at[pl.ds(n * T + t0, SUB)])


def _mm_body(codes_ref, p1_ref, flat_ref, wo_ref, o_ref):
    n = pl.program_id(0)
    k = pl.program_id(1)

    @pl.when((n == 0) & (k == 0))
    def _():
        g_r = lax.broadcasted_iota(jnp.int32, (R, R * ALPHA), 0)
        g_c = lax.broadcasted_iota(jnp.int32, (R, R * ALPHA), 1)
        erep = jnp.where(g_c // ALPHA == g_r, 1.0, 0.0)
        c_rep = jnp.dot(codes_ref[...].astype(jnp.float32), erep)
        a_i = lax.broadcasted_iota(jnp.int32, (T, R * ALPHA), 1) % ALPHA
        onehot = (c_rep.astype(jnp.int32) == a_i).astype(jnp.float32)
        o_ref[...] = jnp.dot(onehot, p1_ref[...],
                             preferred_element_type=jnp.float32)

    a = flat_ref[0]
    t_i = lax.broadcasted_iota(jnp.int32, a.shape, 0)
    a = jnp.where(t_i >= n + 1, a, 0.0)
    o_ref[...] += jnp.dot(a, wo_ref[0], preferred_element_type=jnp.float32)


def kernel(x, Wq, table_1, table_2, table_3, Wo):
    gidx, codes = pl.pallas_call(
        _index_body,
        grid=(NTB,),
        in_specs=[
            pl.BlockSpec((1, TB, D), lambda tb: (0, tb, 0)),
            pl.BlockSpec((C, D), lambda tb: (0, 0)),
        ],
        out_specs=(pl.BlockSpec((NO, TB, R), lambda tb: (0, tb, 0)),
                   pl.BlockSpec((TB, R), lambda tb: (tb, 0))),
        out_shape=(jax.ShapeDtypeStruct((NO, T, R), jnp.int32),
                   jax.ShapeDtypeStruct((T, R), jnp.int32)),
        scratch_shapes=[pltpu.VMEM((8, R), jnp.int32)],
        compiler_params=pltpu.CompilerParams(
            dimension_semantics=("arbitrary",),
        ),
    )(x, Wq.T)
    gflat = gidx.reshape(NO * T * R)

    wo4 = Wo.reshape(3, R, MEM, D)
    p1 = pl.pallas_call(
        _p1_body,
        grid=(1,),
        in_specs=[
            pl.BlockSpec((R * ALPHA, MEM), lambda i: (0, 0)),
            pl.BlockSpec((1, R, MEM, D), lambda i: (0, 0, 0, 0)),
        ],
        out_specs=pl.BlockSpec((R * ALPHA, D), lambda i: (0, 0)),
        out_shape=jax.ShapeDtypeStruct((R * ALPHA, D), jnp.float32),
    )(table_1, wo4)

    mesh = plsc.VectorSubcoreMesh(core_axis_name="c", subcore_axis_name="s")
    sc_scratch = [
        pltpu.VMEM((ROWS,), jnp.int32),
        pltpu.VMEM((ROWS,), jnp.int32),
        pltpu.VMEM((ROWS, MEM), jnp.float32),
        pltpu.VMEM((ROWS, MEM), jnp.float32),
        pltpu.SemaphoreType.DMA,
        pltpu.SemaphoreType.DMA,
    ]

    sc_gather = functools.partial(
        pl.kernel,
        out_type=jax.ShapeDtypeStruct((NO * T, R * MEM), jnp.float32),
        mesh=mesh,
        scratch_types=sc_scratch,
    )(_sc_gather_body)
    rows = sc_gather(table_2, table_3, gflat)
    flat = rows.reshape(NO, T, R * MEM)

    wo3 = Wo.reshape(3, R * MEM, D)
    out = pl.pallas_call(
        _mm_body,
        grid=(NO, NKB),
        in_specs=[
            pl.BlockSpec((T, R), lambda n, k: (0, 0)),
            pl.BlockSpec((R * ALPHA, D), lambda n, k: (0, 0)),
            pl.BlockSpec((1, T, KB), lambda n, k: (n, 0, k)),
            pl.BlockSpec((1, KB, D), lambda n, k: (n + 1, k, 0)),
        ],
        out_specs=pl.BlockSpec((T, D), lambda n, k: (0, 0)),
        out_shape=jax.ShapeDtypeStruct((T, D), jnp.float32),
        compiler_params=pltpu.CompilerParams(
            dimension_semantics=("arbitrary", "arbitrary"),
        ),
    )(codes, p1, flat, wo3)

    return out.reshape(x.shape[0], T, D)
